# Initial kernel scaffold; baseline (speedup 1.0000x reference)
import jax
import jax.numpy as jnp
from jax.experimental import pallas as pl


def kernel(inputs, sampled_edges, msg_fc1_w, msg_fc1_b, msg_fc2_w, msg_fc2_b, hidden_r_w, hidden_i_w, hidden_h_w, input_r_w, input_r_b, input_i_w, input_i_b, input_n_w, input_n_b, out_fc1_w, out_fc1_b, out_fc2_w, out_fc2_b, out_fc3_w, out_fc3_b):
    raise NotImplementedError("write your pallas kernel here")


# speedup vs baseline: 3.7220x; 3.7220x over previous
import jax
import jax.numpy as jnp
import numpy as np
from jax.experimental import pallas as pl
from jax.experimental.pallas import tpu as pltpu


def _decoder_body(T, V, DIN, H, ET,
                  w_ref, ins_ref, w1_ref, b1_ref, w2_ref, b2_ref,
                  hr_ref, hi_ref, hh_ref,
                  irw_ref, irb_ref, iiw_ref, iib_ref, inw_ref, inb_ref,
                  o1w_ref, o1b_ref, o2w_ref, o2b_ref, o3w_ref, o3b_ref,
                  out_ref):
    P = V * V
    f32 = jnp.float32

    pr = jax.lax.broadcasted_iota(jnp.int32, (P, 2 * V), 0)
    pc = jax.lax.broadcasted_iota(jnp.int32, (P, 2 * V), 1)
    g2 = (((pr // V) == pc).astype(f32) +
          ((pr % V) == (pc - V)).astype(f32))
    vr = jax.lax.broadcasted_iota(jnp.int32, (V, P), 0)
    vc = jax.lax.broadcasted_iota(jnp.int32, (V, P), 1)
    g_recv_t = (vr == (vc % V)).astype(f32)

    w = w_ref[0]
    ins = ins_ref[0]

    inv_norm = 1.0 / ((ET - 1.0) * (V - 1.0))
    hidden = jnp.zeros((V, H), dtype=f32)

    dot = lambda a, b: jnp.dot(a, b, preferred_element_type=f32)

    for t in range(T):
        m2w = jnp.zeros((P, H), dtype=f32)
        for et in range(1, ET):
            s_part = dot(hidden, w1_ref[et, H:, :])
            a_part = dot(hidden, w1_ref[et, :H, :]) + b1_ref[et]
            sa = jnp.concatenate([s_part, a_part], axis=0)
            pre = dot(g2, sa)
            m = jnp.tanh(pre)
            m2 = jnp.tanh(dot(m, w2_ref[et]) + b2_ref[et])
            m2w = m2w + m2 * w[:, et:et + 1]
        agg = dot(g_recv_t, m2w) * inv_norm

        inp_r = dot(ins, irw_ref[...]) + irb_ref[0]
        inp_i = dot(ins, iiw_ref[...]) + iib_ref[0]
        inp_n = dot(ins, inw_ref[...]) + inb_ref[0]
        r = jax.nn.sigmoid(inp_r + dot(agg, hr_ref[...]))
        ig = jax.nn.sigmoid(inp_i + dot(agg, hi_ref[...]))
        n = jnp.tanh(inp_n + r * dot(agg, hh_ref[...]))
        hidden = (1.0 - ig) * n + ig * hidden

        p = jax.nn.relu(dot(hidden, o1w_ref[...]) + o1b_ref[0])
        p = jax.nn.relu(dot(p, o2w_ref[...]) + o2b_ref[0])
        p = dot(p, o3w_ref[...]) + o3b_ref[0]
        pred = ins + p
        out_ref[0, t] = pred
        ins = pred


def kernel(inputs, sampled_edges, msg_fc1_w, msg_fc1_b, msg_fc2_w,
           msg_fc2_b, hidden_r_w, hidden_i_w, hidden_h_w, input_r_w,
           input_r_b, input_i_w, input_i_b, input_n_w, input_n_b,
           out_fc1_w, out_fc1_b, out_fc2_w, out_fc2_b, out_fc3_w,
           out_fc3_b):
    B, T, V, DIN = inputs.shape
    H = hidden_r_w.shape[0]
    ET = msg_fc1_w.shape[0]
    P = V * V

    adj = np.ones((V, V)) - np.eye(V)
    send_np, recv_np = np.where(adj)
    p_idx = jnp.asarray(send_np * V + recv_np, dtype=jnp.int32)
    w_dense = jnp.zeros((B, P, ET), dtype=jnp.float32)
    w_dense = w_dense.at[:, p_idx, :].set(sampled_edges)

    ins0 = inputs[:, 0]

    def body(*refs):
        _decoder_body(T, V, DIN, H, ET, *refs)

    rep3 = lambda shp: pl.BlockSpec(shp, lambda b: (0, 0, 0))
    rep2 = lambda shp: pl.BlockSpec(shp, lambda b: (0, 0))

    out = pl.pallas_call(
        body,
        grid=(B,),
        in_specs=[
            pl.BlockSpec((1, P, ET), lambda b: (b, 0, 0)),
            pl.BlockSpec((1, V, DIN), lambda b: (b, 0, 0)),
            rep3((ET, 2 * H, H)),
            rep2((ET, H)),
            rep3((ET, H, H)),
            rep2((ET, H)),
            rep2((H, H)), rep2((H, H)), rep2((H, H)),
            rep2((DIN, H)), rep2((1, H)),
            rep2((DIN, H)), rep2((1, H)),
            rep2((DIN, H)), rep2((1, H)),
            rep2((H, H)), rep2((1, H)),
            rep2((H, H)), rep2((1, H)),
            rep2((H, DIN)), rep2((1, DIN)),
        ],
        out_specs=pl.BlockSpec((1, T, V, DIN), lambda b: (b, 0, 0, 0)),
        out_shape=jax.ShapeDtypeStruct((B, T, V, DIN), jnp.float32),
        compiler_params=pltpu.CompilerParams(
            dimension_semantics=("parallel",)),
    )(w_dense, ins0, msg_fc1_w, msg_fc1_b, msg_fc2_w, msg_fc2_b,
      hidden_r_w, hidden_i_w, hidden_h_w,
      input_r_w, input_r_b.reshape(1, H),
      input_i_w, input_i_b.reshape(1, H),
      input_n_w, input_n_b.reshape(1, H),
      out_fc1_w, out_fc1_b.reshape(1, H),
      out_fc2_w, out_fc2_b.reshape(1, H),
      out_fc3_w, out_fc3_b.reshape(1, DIN))
    return out


# speedup vs baseline: 4.7840x; 1.2853x over previous
import jax
import jax.numpy as jnp
import numpy as np
from jax.experimental import pallas as pl
from jax.experimental.pallas import tpu as pltpu


def _decoder_body(T, V, Vp, DIN, H, ET,
                  w_ref, ins_ref, w1_ref, b1_ref, w2_ref, b2_ref,
                  hr_ref, hi_ref, hh_ref,
                  irw_ref, irb_ref, iiw_ref, iib_ref, inw_ref, inb_ref,
                  o1w_ref, o1b_ref, o2w_ref, o2b_ref, o3w_ref, o3b_ref,
                  out_ref):
    P = V * Vp
    f32 = jnp.float32

    w = w_ref[0]
    ins = ins_ref[0]

    inv_norm = 1.0 / ((ET - 1.0) * (V - 1.0))
    hidden = jnp.zeros((Vp, H), dtype=f32)

    dot = lambda a, b: jnp.dot(a, b, preferred_element_type=f32)

    for t in range(T):
        m2w = jnp.zeros((P, H), dtype=f32)
        for et in range(1, ET):
            s_part = dot(hidden, w1_ref[et, H:, :])
            a_part = dot(hidden, w1_ref[et, :H, :]) + b1_ref[et]
            pre = s_part[:V][:, None, :] + a_part[None, :, :]
            m = jnp.tanh(pre).reshape(P, H)
            m2 = jnp.tanh(dot(m, w2_ref[et]) + b2_ref[et])
            m2w = m2w + m2 * w[:, et:et + 1]
        agg = jnp.sum(m2w.reshape(V, Vp, H), axis=0) * inv_norm

        inp_r = dot(ins, irw_ref[...]) + irb_ref[0]
        inp_i = dot(ins, iiw_ref[...]) + iib_ref[0]
        inp_n = dot(ins, inw_ref[...]) + inb_ref[0]
        r = jax.nn.sigmoid(inp_r + dot(agg, hr_ref[...]))
        ig = jax.nn.sigmoid(inp_i + dot(agg, hi_ref[...]))
        n = jnp.tanh(inp_n + r * dot(agg, hh_ref[...]))
        hidden = (1.0 - ig) * n + ig * hidden

        p = jax.nn.relu(dot(hidden, o1w_ref[...]) + o1b_ref[0])
        p = jax.nn.relu(dot(p, o2w_ref[...]) + o2b_ref[0])
        p = dot(p, o3w_ref[...]) + o3b_ref[0]
        pred = ins + p
        out_ref[0, t] = pred[:V]
        ins = pred


def kernel(inputs, sampled_edges, msg_fc1_w, msg_fc1_b, msg_fc2_w,
           msg_fc2_b, hidden_r_w, hidden_i_w, hidden_h_w, input_r_w,
           input_r_b, input_i_w, input_i_b, input_n_w, input_n_b,
           out_fc1_w, out_fc1_b, out_fc2_w, out_fc2_b, out_fc3_w,
           out_fc3_b):
    B, T, V, DIN = inputs.shape
    H = hidden_r_w.shape[0]
    ET = msg_fc1_w.shape[0]
    Vp = (V + 7) // 8 * 8
    P = V * Vp

    adj = np.ones((V, V)) - np.eye(V)
    send_np, recv_np = np.where(adj)
    p_idx = jnp.asarray(send_np * Vp + recv_np, dtype=jnp.int32)
    w_dense = jnp.zeros((B, P, ET), dtype=jnp.float32)
    w_dense = w_dense.at[:, p_idx, :].set(sampled_edges)

    ins0 = jnp.pad(inputs[:, 0], ((0, 0), (0, Vp - V), (0, 0)))

    def body(*refs):
        _decoder_body(T, V, Vp, DIN, H, ET, *refs)

    rep3 = lambda shp: pl.BlockSpec(shp, lambda b: (0, 0, 0))
    rep2 = lambda shp: pl.BlockSpec(shp, lambda b: (0, 0))

    out = pl.pallas_call(
        body,
        grid=(B,),
        in_specs=[
            pl.BlockSpec((1, P, ET), lambda b: (b, 0, 0)),
            pl.BlockSpec((1, Vp, DIN), lambda b: (b, 0, 0)),
            rep3((ET, 2 * H, H)),
            rep2((ET, H)),
            rep3((ET, H, H)),
            rep2((ET, H)),
            rep2((H, H)), rep2((H, H)), rep2((H, H)),
            rep2((DIN, H)), rep2((1, H)),
            rep2((DIN, H)), rep2((1, H)),
            rep2((DIN, H)), rep2((1, H)),
            rep2((H, H)), rep2((1, H)),
            rep2((H, H)), rep2((1, H)),
            rep2((H, DIN)), rep2((1, DIN)),
        ],
        out_specs=pl.BlockSpec((1, T, V, DIN), lambda b: (b, 0, 0, 0)),
        out_shape=jax.ShapeDtypeStruct((B, T, V, DIN), jnp.float32),
        compiler_params=pltpu.CompilerParams(
            dimension_semantics=("parallel",)),
    )(w_dense, ins0, msg_fc1_w, msg_fc1_b, msg_fc2_w, msg_fc2_b,
      hidden_r_w, hidden_i_w, hidden_h_w,
      input_r_w, input_r_b.reshape(1, H),
      input_i_w, input_i_b.reshape(1, H),
      input_n_w, input_n_b.reshape(1, H),
      out_fc1_w, out_fc1_b.reshape(1, H),
      out_fc2_w, out_fc2_b.reshape(1, H),
      out_fc3_w, out_fc3_b.reshape(1, DIN))
    return out
